# Initial kernel scaffold; baseline (speedup 1.0000x reference)
import jax
import jax.numpy as jnp
from jax.experimental import pallas as pl


def kernel(x, pos_embed, scale):
    raise NotImplementedError("write your pallas kernel here")


# speedup vs baseline: 3.3725x; 3.3725x over previous
import jax
import jax.numpy as jnp
from jax.experimental import pallas as pl
from jax.experimental.pallas import tpu as pltpu

BLOCK_S = 1024


def _body(scale_ref, x_ref, pos_ref, out_ref):
    out_ref[0] = x_ref[0] + pos_ref[...] * scale_ref[0]


def kernel(x, pos_embed, scale):
    B, S, D = x.shape
    num_s = S // BLOCK_S

    grid_spec = pltpu.PrefetchScalarGridSpec(
        num_scalar_prefetch=1,
        grid=(num_s, B),
        in_specs=[
            pl.BlockSpec((1, BLOCK_S, D), lambda s, b, *_: (b, s, 0)),
            pl.BlockSpec((BLOCK_S, D), lambda s, b, *_: (s, 0)),
        ],
        out_specs=pl.BlockSpec((1, BLOCK_S, D), lambda s, b, *_: (b, s, 0)),
    )

    return pl.pallas_call(
        _body,
        grid_spec=grid_spec,
        out_shape=jax.ShapeDtypeStruct((B, S, D), x.dtype),
        compiler_params=pltpu.CompilerParams(
            dimension_semantics=("arbitrary", "arbitrary"),
        ),
    )(scale, x, pos_embed[:S])


# speedup vs baseline: 3.6099x; 1.0704x over previous
import jax
import jax.numpy as jnp
from jax.experimental import pallas as pl
from jax.experimental.pallas import tpu as pltpu

BLOCK_S = 2048


def _body(scale_ref, x_ref, pos_ref, out_ref):
    out_ref[0] = x_ref[0] + pos_ref[...] * scale_ref[0]


def kernel(x, pos_embed, scale):
    B, S, D = x.shape
    num_s = S // BLOCK_S

    grid_spec = pltpu.PrefetchScalarGridSpec(
        num_scalar_prefetch=1,
        grid=(num_s, B),
        in_specs=[
            pl.BlockSpec((1, BLOCK_S, D), lambda s, b, *_: (b, s, 0)),
            pl.BlockSpec((BLOCK_S, D), lambda s, b, *_: (s, 0)),
        ],
        out_specs=pl.BlockSpec((1, BLOCK_S, D), lambda s, b, *_: (b, s, 0)),
    )

    return pl.pallas_call(
        _body,
        grid_spec=grid_spec,
        out_shape=jax.ShapeDtypeStruct((B, S, D), x.dtype),
        compiler_params=pltpu.CompilerParams(
            dimension_semantics=("arbitrary", "arbitrary"),
        ),
    )(scale, x, pos_embed[:S])
